# baseline (speedup 1.0000x reference)
import functools

import jax
import jax.numpy as jnp
from jax import lax
from jax.experimental import pallas as pl
from jax.experimental.pallas import tpu as pltpu
from jax.experimental.pallas import tpu_sc as plsc

EMBED = 64
LANES = 16
NCORES = 2
NSUB = 16
NWORKERS = NCORES * NSUB
CHUNK = 128


def _sc_scores(uidx, pidx, nidx, users_int, users_pop, items_int, items_pop, q, b):
    B = uidx.shape[0]
    per_w = B // NWORKERS
    n_chunks = per_w // CHUNK
    out = jax.ShapeDtypeStruct((B,), jnp.float32)
    mesh = plsc.VectorSubcoreMesh(
        core_axis_name="c", subcore_axis_name="s",
        num_cores=NCORES, num_subcores=NSUB)

    @functools.partial(
        pl.kernel,
        out_type=[out] * 8,
        mesh=mesh,
        compiler_params=pltpu.CompilerParams(
            needs_layout_passes=False, use_tc_tiling_on_sc=False),
        scratch_types=[
            pltpu.VMEM((CHUNK,), jnp.int32),
            pltpu.VMEM((CHUNK,), jnp.int32),
            pltpu.VMEM((CHUNK,), jnp.int32),
            pltpu.VMEM((CHUNK, EMBED), jnp.float32),
            pltpu.VMEM((CHUNK, EMBED), jnp.float32),
            pltpu.VMEM((CHUNK, EMBED), jnp.float32),
            pltpu.VMEM((CHUNK, EMBED), jnp.float32),
            pltpu.VMEM((CHUNK, EMBED), jnp.float32),
            pltpu.VMEM((CHUNK, EMBED), jnp.float32),
            pltpu.VMEM((CHUNK,), jnp.float32),
            pltpu.VMEM((CHUNK,), jnp.float32),
            pltpu.VMEM((CHUNK,), jnp.float32),
            pltpu.VMEM((CHUNK,), jnp.float32),
            pltpu.VMEM((CHUNK,), jnp.float32),
            pltpu.VMEM((CHUNK,), jnp.float32),
            pltpu.VMEM((CHUNK,), jnp.float32),
            pltpu.VMEM((CHUNK,), jnp.float32),
            pltpu.SemaphoreType.DMA,
        ],
    )
    def k(uidx_h, pidx_h, nidx_h, uint_h, upop_h, iint_h, ipop_h, q_h, b_h,
          o_pint, o_nint, o_ppop, o_npop, o_qp, o_bp, o_qn, o_bn,
          uidx_v, pidx_v, nidx_v, uiv, upv, pivv, ppv, niv, npv,
          qp_v, bp_v, qn_v, bn_v, spi, sni, spp, snp, sem):
        wid = lax.axis_index("s") * NCORES + lax.axis_index("c")
        base = wid * per_w

        def chunk(g, _):
            off = base + g * CHUNK
            pltpu.sync_copy(uidx_h.at[pl.ds(off, CHUNK)], uidx_v)
            pltpu.sync_copy(pidx_h.at[pl.ds(off, CHUNK)], pidx_v)
            pltpu.sync_copy(nidx_h.at[pl.ds(off, CHUNK)], nidx_v)
            cps = [
                pltpu.async_copy(uint_h.at[uidx_v], uiv, sem),
                pltpu.async_copy(upop_h.at[uidx_v], upv, sem),
                pltpu.async_copy(iint_h.at[pidx_v], pivv, sem),
                pltpu.async_copy(ipop_h.at[pidx_v], ppv, sem),
                pltpu.async_copy(iint_h.at[nidx_v], niv, sem),
                pltpu.async_copy(ipop_h.at[nidx_v], npv, sem),
                pltpu.async_copy(q_h.at[pidx_v], qp_v, sem),
                pltpu.async_copy(b_h.at[pidx_v], bp_v, sem),
                pltpu.async_copy(q_h.at[nidx_v], qn_v, sem),
                pltpu.async_copy(b_h.at[nidx_v], bn_v, sem),
            ]
            for cp in cps:
                cp.wait()

            iota = lax.iota(jnp.int32, LANES)

            def grp(j, _):
                rows = j * LANES + iota
                pi = jnp.zeros((LANES,), jnp.float32)
                ni = jnp.zeros((LANES,), jnp.float32)
                pp = jnp.zeros((LANES,), jnp.float32)
                np_ = jnp.zeros((LANES,), jnp.float32)
                for d in range(EMBED):
                    cols = jnp.full((LANES,), d, jnp.int32)
                    ui = plsc.load_gather(uiv, [rows, cols])
                    up = plsc.load_gather(upv, [rows, cols])
                    pi = pi + ui * plsc.load_gather(pivv, [rows, cols])
                    ni = ni + ui * plsc.load_gather(niv, [rows, cols])
                    pp = pp + up * plsc.load_gather(ppv, [rows, cols])
                    np_ = np_ + up * plsc.load_gather(npv, [rows, cols])
                base16 = j * LANES
                spi[pl.ds(base16, LANES)] = pi
                sni[pl.ds(base16, LANES)] = ni
                spp[pl.ds(base16, LANES)] = pp
                snp[pl.ds(base16, LANES)] = np_
                return 0

            lax.fori_loop(0, CHUNK // LANES, grp, 0)

            pltpu.sync_copy(spi, o_pint.at[pl.ds(off, CHUNK)])
            pltpu.sync_copy(sni, o_nint.at[pl.ds(off, CHUNK)])
            pltpu.sync_copy(spp, o_ppop.at[pl.ds(off, CHUNK)])
            pltpu.sync_copy(snp, o_npop.at[pl.ds(off, CHUNK)])
            pltpu.sync_copy(qp_v, o_qp.at[pl.ds(off, CHUNK)])
            pltpu.sync_copy(bp_v, o_bp.at[pl.ds(off, CHUNK)])
            pltpu.sync_copy(qn_v, o_qn.at[pl.ds(off, CHUNK)])
            pltpu.sync_copy(bn_v, o_bn.at[pl.ds(off, CHUNK)])
            return 0

        lax.fori_loop(0, n_chunks, chunk, 0)

    return k(uidx, pidx, nidx, users_int, users_pop, items_int, items_pop, q, b)


def _log_sigmoid(x):
    return jnp.minimum(x, 0.0) - jnp.log1p(jnp.exp(-jnp.abs(x)))


def _softplus(x):
    return jnp.maximum(x, 0.0) + jnp.log1p(jnp.exp(-jnp.abs(x)))


def _loss_body(pi_r, ni_r, pp_r, np_r, qp_r, bp_r, qn_r, bn_r, m_r, o_r):
    pi = pi_r[...]
    ni = ni_r[...]
    pp = pp_r[...]
    np_ = np_r[...]
    m = m_r[...]
    inv_b = 1.0 / pi.size
    loss_int = -jnp.sum(m * _log_sigmoid(pi - ni)) * inv_b
    loss_pop = -(jnp.sum(m * _log_sigmoid(np_ - pp))
                 + jnp.sum((1.0 - m) * _log_sigmoid(pp - np_))) * inv_b
    pop_p = _softplus(qp_r[...]) + _softplus(bp_r[...])
    pop_n = _softplus(qn_r[...]) + _softplus(bn_r[...])
    p_tide = jnp.tanh(pop_p) * (pi + pp)
    n_tide = jnp.tanh(pop_n) * (ni + np_)
    loss_tide = -jnp.sum(_log_sigmoid(p_tide - n_tide)) * inv_b
    total = 0.1 * loss_int + 0.1 * loss_pop + 0.2 * loss_tide
    o_r[...] = jnp.reshape(total, (1, 1))


def _loss_tc(pi, ni, pp, np_, qp, bp, qn, bn, maskf):
    return pl.pallas_call(
        _loss_body,
        out_shape=jax.ShapeDtypeStruct((1, 1), jnp.float32),
    )(pi, ni, pp, np_, qp, bp, qn, bn, maskf)


def kernel(user, item_p, item_n, mask, users_int, users_pop, items_int, items_pop, q, b):
    uidx = user.reshape(-1)
    pidx = item_p.reshape(-1)
    nidx = item_n.reshape(-1)
    outs = _sc_scores(uidx, pidx, nidx, users_int, users_pop,
                      items_int, items_pop, q, b)
    rows = uidx.shape[0] // 128
    rs = lambda x: x.reshape(rows, 128)
    maskf = rs(mask.reshape(-1).astype(jnp.float32))
    loss = _loss_tc(*(rs(o) for o in outs), maskf)
    return loss.reshape(())


# speedup vs baseline: 1.2267x; 1.2267x over previous
import functools

import jax
import jax.numpy as jnp
from jax import lax
from jax.experimental import pallas as pl
from jax.experimental.pallas import tpu as pltpu
from jax.experimental.pallas import tpu_sc as plsc

EMBED = 64
UCOLS = 2 * EMBED
ICOLS = 2 * EMBED + 8
LANES = 16
NCORES = 2
NSUB = 16
NWORKERS = NCORES * NSUB
CHUNK = 64


def _sc_scores(uidx, pidx, nidx, u_comb, i_comb):
    B = uidx.shape[0]
    per_w = B // NWORKERS
    n_chunks = per_w // CHUNK
    assert n_chunks % 2 == 0
    out = jax.ShapeDtypeStruct((B,), jnp.float32)
    mesh = plsc.VectorSubcoreMesh(
        core_axis_name="c", subcore_axis_name="s",
        num_cores=NCORES, num_subcores=NSUB)

    ubuf = pltpu.VMEM((CHUNK, UCOLS), jnp.float32)
    ibuf = pltpu.VMEM((CHUNK, ICOLS), jnp.float32)
    stage = pltpu.VMEM((per_w,), jnp.float32)

    @functools.partial(
        pl.kernel,
        out_type=[out] * 8,
        mesh=mesh,
        compiler_params=pltpu.CompilerParams(
            needs_layout_passes=False, use_tc_tiling_on_sc=False),
        scratch_types=[
            pltpu.VMEM((per_w,), jnp.int32),
            pltpu.VMEM((per_w,), jnp.int32),
            pltpu.VMEM((per_w,), jnp.int32),
            [ubuf, ibuf, ibuf],
            [ubuf, ibuf, ibuf],
            [stage] * 8,
            pltpu.SemaphoreType.DMA,
        ],
    )
    def k(uidx_h, pidx_h, nidx_h, ucomb_h, icomb_h,
          o_pint, o_nint, o_ppop, o_npop, o_qp, o_bp, o_qn, o_bn,
          idx_u, idx_p, idx_n, bufa, bufb, st, sem_r):
        wid = lax.axis_index("s") * NCORES + lax.axis_index("c")
        base = wid * per_w
        pltpu.sync_copy(uidx_h.at[pl.ds(base, per_w)], idx_u)
        pltpu.sync_copy(pidx_h.at[pl.ds(base, per_w)], idx_p)
        pltpu.sync_copy(nidx_h.at[pl.ds(base, per_w)], idx_n)

        def issue_rows(g, bufs):
            s = pl.ds(g * CHUNK, CHUNK)
            pltpu.async_copy(ucomb_h.at[idx_u.at[s]], bufs[0], sem_r)
            pltpu.async_copy(icomb_h.at[idx_p.at[s]], bufs[1], sem_r)
            pltpu.async_copy(icomb_h.at[idx_n.at[s]], bufs[2], sem_r)

        def wait_rows(bufs):
            s0 = pl.ds(0, CHUNK)
            pltpu.make_async_copy(
                ucomb_h.at[idx_u.at[s0]], bufs[0], sem_r).wait()
            pltpu.make_async_copy(
                icomb_h.at[idx_p.at[s0]], bufs[1], sem_r).wait()
            pltpu.make_async_copy(
                icomb_h.at[idx_n.at[s0]], bufs[2], sem_r).wait()

        iota = lax.iota(jnp.int32, LANES)

        def compute(g, bufs):
            ub, pb, nb = bufs

            def grp(j, _):
                rows = j * LANES + iota
                pi = jnp.zeros((LANES,), jnp.float32)
                ni = jnp.zeros((LANES,), jnp.float32)
                pp = jnp.zeros((LANES,), jnp.float32)
                np_ = jnp.zeros((LANES,), jnp.float32)
                for d in range(EMBED):
                    ci = jnp.full((LANES,), d, jnp.int32)
                    cp = jnp.full((LANES,), EMBED + d, jnp.int32)
                    ui = plsc.load_gather(ub, [rows, ci])
                    up = plsc.load_gather(ub, [rows, cp])
                    pi = pi + ui * plsc.load_gather(pb, [rows, ci])
                    ni = ni + ui * plsc.load_gather(nb, [rows, ci])
                    pp = pp + up * plsc.load_gather(pb, [rows, cp])
                    np_ = np_ + up * plsc.load_gather(nb, [rows, cp])
                cq = jnp.full((LANES,), 2 * EMBED, jnp.int32)
                cb = jnp.full((LANES,), 2 * EMBED + 1, jnp.int32)
                o = g * CHUNK + j * LANES
                st[0][pl.ds(o, LANES)] = pi
                st[1][pl.ds(o, LANES)] = ni
                st[2][pl.ds(o, LANES)] = pp
                st[3][pl.ds(o, LANES)] = np_
                st[4][pl.ds(o, LANES)] = plsc.load_gather(pb, [rows, cq])
                st[5][pl.ds(o, LANES)] = plsc.load_gather(pb, [rows, cb])
                st[6][pl.ds(o, LANES)] = plsc.load_gather(nb, [rows, cq])
                st[7][pl.ds(o, LANES)] = plsc.load_gather(nb, [rows, cb])
                return 0

            lax.fori_loop(0, CHUNK // LANES, grp, 0)

        issue_rows(0, bufa)

        def pair(t, _):
            g0 = 2 * t
            g1 = 2 * t + 1
            wait_rows(bufa)
            issue_rows(g1, bufb)
            compute(g0, bufa)
            wait_rows(bufb)

            @pl.when(g1 + 1 < n_chunks)
            def _():
                issue_rows(g1 + 1, bufa)

            compute(g1, bufb)
            return 0

        lax.fori_loop(0, n_chunks // 2, pair, 0)

        outs = (o_pint, o_nint, o_ppop, o_npop, o_qp, o_bp, o_qn, o_bn)
        for s, o in zip(st, outs):
            pltpu.sync_copy(s, o.at[pl.ds(base, per_w)])

    return k(uidx, pidx, nidx, u_comb, i_comb)


def _log_sigmoid(x):
    return jnp.minimum(x, 0.0) - jnp.log1p(jnp.exp(-jnp.abs(x)))


def _softplus(x):
    return jnp.maximum(x, 0.0) + jnp.log1p(jnp.exp(-jnp.abs(x)))


def _loss_body(pi_r, ni_r, pp_r, np_r, qp_r, bp_r, qn_r, bn_r, m_r, o_r):
    pi = pi_r[...]
    ni = ni_r[...]
    pp = pp_r[...]
    np_ = np_r[...]
    m = m_r[...]
    inv_b = 1.0 / pi.size
    loss_int = -jnp.sum(m * _log_sigmoid(pi - ni)) * inv_b
    loss_pop = -(jnp.sum(m * _log_sigmoid(np_ - pp))
                 + jnp.sum((1.0 - m) * _log_sigmoid(pp - np_))) * inv_b
    pop_p = _softplus(qp_r[...]) + _softplus(bp_r[...])
    pop_n = _softplus(qn_r[...]) + _softplus(bn_r[...])
    p_tide = jnp.tanh(pop_p) * (pi + pp)
    n_tide = jnp.tanh(pop_n) * (ni + np_)
    loss_tide = -jnp.sum(_log_sigmoid(p_tide - n_tide)) * inv_b
    total = 0.1 * loss_int + 0.1 * loss_pop + 0.2 * loss_tide
    o_r[...] = jnp.reshape(total, (1, 1))


def _loss_tc(pi, ni, pp, np_, qp, bp, qn, bn, maskf):
    return pl.pallas_call(
        _loss_body,
        out_shape=jax.ShapeDtypeStruct((1, 1), jnp.float32),
    )(pi, ni, pp, np_, qp, bp, qn, bn, maskf)


def kernel(user, item_p, item_n, mask, users_int, users_pop, items_int, items_pop, q, b):
    uidx = user.reshape(-1)
    pidx = item_p.reshape(-1)
    nidx = item_n.reshape(-1)
    n_items = q.shape[0]
    u_comb = jnp.concatenate([users_int, users_pop], axis=1)
    i_comb = jnp.concatenate(
        [items_int, items_pop, q[:, None], b[:, None],
         jnp.zeros((n_items, ICOLS - 2 * EMBED - 2), jnp.float32)], axis=1)
    outs = _sc_scores(uidx, pidx, nidx, u_comb, i_comb)
    rows = uidx.shape[0] // 128
    rs = lambda x: x.reshape(rows, 128)
    maskf = rs(mask.reshape(-1).astype(jnp.float32))
    loss = _loss_tc(*(rs(o) for o in outs), maskf)
    return loss.reshape(())


# speedup vs baseline: 1.2450x; 1.0149x over previous
import functools

import jax
import jax.numpy as jnp
from jax import lax
from jax.experimental import pallas as pl
from jax.experimental.pallas import tpu as pltpu
from jax.experimental.pallas import tpu_sc as plsc

EMBED = 64
COMB = 2 * EMBED
LANES = 16
NCORES = 2
NSUB = 16
NWORKERS = NCORES * NSUB
CHUNK = 64


def _sc_scores(uidx, pidx, nidx, u_comb, i_comb, w):
    B = uidx.shape[0]
    per_w = B // NWORKERS
    n_chunks = per_w // CHUNK
    assert n_chunks % 2 == 0
    out = jax.ShapeDtypeStruct((B,), jnp.float32)
    mesh = plsc.VectorSubcoreMesh(
        core_axis_name="c", subcore_axis_name="s",
        num_cores=NCORES, num_subcores=NSUB)

    rbuf = pltpu.VMEM((CHUNK, COMB), jnp.float32)
    stage = pltpu.VMEM((per_w,), jnp.float32)

    @functools.partial(
        pl.kernel,
        out_type=[out] * 6,
        mesh=mesh,
        compiler_params=pltpu.CompilerParams(
            needs_layout_passes=False, use_tc_tiling_on_sc=True),
        scratch_types=[
            pltpu.VMEM((per_w,), jnp.int32),
            pltpu.VMEM((per_w,), jnp.int32),
            pltpu.VMEM((per_w,), jnp.int32),
            [rbuf] * 3,
            [rbuf] * 3,
            [stage] * 6,
            pltpu.SemaphoreType.DMA,
            pltpu.SemaphoreType.DMA,
        ],
    )
    def k(uidx_h, pidx_h, nidx_h, ucomb_h, icomb_h, w_h,
          o_pint, o_nint, o_ppop, o_npop, o_wp, o_wn,
          idx_u, idx_p, idx_n, bufa, bufb, st, sem_r, sem_s):
        wid = lax.axis_index("s") * NCORES + lax.axis_index("c")
        base = wid * per_w
        pltpu.sync_copy(uidx_h.at[pl.ds(base, per_w)], idx_u)
        pltpu.sync_copy(pidx_h.at[pl.ds(base, per_w)], idx_p)
        pltpu.sync_copy(nidx_h.at[pl.ds(base, per_w)], idx_n)

        pltpu.async_copy(w_h.at[idx_p], st[4], sem_s)
        pltpu.async_copy(w_h.at[idx_n], st[5], sem_s)

        def issue_rows(g, bufs):
            s = pl.ds(g * CHUNK, CHUNK)
            pltpu.async_copy(ucomb_h.at[idx_u.at[s]], bufs[0], sem_r)
            pltpu.async_copy(icomb_h.at[idx_p.at[s]], bufs[1], sem_r)
            pltpu.async_copy(icomb_h.at[idx_n.at[s]], bufs[2], sem_r)

        def wait_rows(bufs):
            s0 = pl.ds(0, CHUNK)
            pltpu.make_async_copy(
                ucomb_h.at[idx_u.at[s0]], bufs[0], sem_r).wait()
            pltpu.make_async_copy(
                icomb_h.at[idx_p.at[s0]], bufs[1], sem_r).wait()
            pltpu.make_async_copy(
                icomb_h.at[idx_n.at[s0]], bufs[2], sem_r).wait()

        iota = lax.iota(jnp.int32, LANES)

        def compute(g, bufs):
            ub, pb, nb = bufs

            def grp(j, _):
                rows = j * LANES + iota
                pi = jnp.zeros((LANES,), jnp.float32)
                ni = jnp.zeros((LANES,), jnp.float32)
                pp = jnp.zeros((LANES,), jnp.float32)
                np_ = jnp.zeros((LANES,), jnp.float32)
                for d in range(EMBED):
                    ci = jnp.full((LANES,), d, jnp.int32)
                    cp = jnp.full((LANES,), EMBED + d, jnp.int32)
                    ui = plsc.load_gather(ub, [rows, ci])
                    up = plsc.load_gather(ub, [rows, cp])
                    pi = pi + ui * plsc.load_gather(pb, [rows, ci])
                    ni = ni + ui * plsc.load_gather(nb, [rows, ci])
                    pp = pp + up * plsc.load_gather(pb, [rows, cp])
                    np_ = np_ + up * plsc.load_gather(nb, [rows, cp])
                o = g * CHUNK + j * LANES
                st[0][pl.ds(o, LANES)] = pi
                st[1][pl.ds(o, LANES)] = ni
                st[2][pl.ds(o, LANES)] = pp
                st[3][pl.ds(o, LANES)] = np_
                return 0

            lax.fori_loop(0, CHUNK // LANES, grp, 0)

        issue_rows(0, bufa)

        def pair(t, _):
            g0 = 2 * t
            g1 = 2 * t + 1
            wait_rows(bufa)
            issue_rows(g1, bufb)
            compute(g0, bufa)
            wait_rows(bufb)

            @pl.when(g1 + 1 < n_chunks)
            def _():
                issue_rows(g1 + 1, bufa)

            compute(g1, bufb)
            return 0

        lax.fori_loop(0, n_chunks // 2, pair, 0)

        pltpu.make_async_copy(w_h.at[idx_p], st[4], sem_s).wait()
        pltpu.make_async_copy(w_h.at[idx_n], st[5], sem_s).wait()

        outs = (o_pint, o_nint, o_ppop, o_npop, o_wp, o_wn)
        for s, o in zip(st, outs):
            pltpu.sync_copy(s, o.at[pl.ds(base, per_w)])

    return k(uidx, pidx, nidx, u_comb, i_comb, w)


def _log_sigmoid(x):
    return jnp.minimum(x, 0.0) - jnp.log1p(jnp.exp(-jnp.abs(x)))


def _softplus(x):
    return jnp.maximum(x, 0.0) + jnp.log1p(jnp.exp(-jnp.abs(x)))


def _loss_body(pi_r, ni_r, pp_r, np_r, wp_r, wn_r, m_r, o_r):
    pi = pi_r[...]
    ni = ni_r[...]
    pp = pp_r[...]
    np_ = np_r[...]
    m = m_r[...]
    inv_b = 1.0 / pi.size
    loss_int = -jnp.sum(m * _log_sigmoid(pi - ni)) * inv_b
    loss_pop = -(jnp.sum(m * _log_sigmoid(np_ - pp))
                 + jnp.sum((1.0 - m) * _log_sigmoid(pp - np_))) * inv_b
    p_tide = wp_r[...] * (pi + pp)
    n_tide = wn_r[...] * (ni + np_)
    loss_tide = -jnp.sum(_log_sigmoid(p_tide - n_tide)) * inv_b
    total = 0.1 * loss_int + 0.1 * loss_pop + 0.2 * loss_tide
    o_r[...] = jnp.reshape(total, (1, 1))


def _loss_tc(pi, ni, pp, np_, wp, wn, maskf):
    return pl.pallas_call(
        _loss_body,
        out_shape=jax.ShapeDtypeStruct((1, 1), jnp.float32),
    )(pi, ni, pp, np_, wp, wn, maskf)


def kernel(user, item_p, item_n, mask, users_int, users_pop, items_int, items_pop, q, b):
    uidx = user.reshape(-1)
    pidx = item_p.reshape(-1)
    nidx = item_n.reshape(-1)
    u_comb = jnp.concatenate([users_int, users_pop], axis=1)
    i_comb = jnp.concatenate([items_int, items_pop], axis=1)
    w = jnp.tanh(_softplus(q) + _softplus(b))
    outs = _sc_scores(uidx, pidx, nidx, u_comb, i_comb, w)
    rows = uidx.shape[0] // 128
    rs = lambda x: x.reshape(rows, 128)
    maskf = rs(mask.reshape(-1).astype(jnp.float32))
    loss = _loss_tc(*(rs(o) for o in outs), maskf)
    return loss.reshape(())


# speedup vs baseline: 1.6115x; 1.2944x over previous
import functools

import jax
import jax.numpy as jnp
from jax import lax
from jax.experimental import pallas as pl
from jax.experimental.pallas import tpu as pltpu
from jax.experimental.pallas import tpu_sc as plsc

EMBED = 64
COMB = 2 * EMBED
NCORES = 2
NSUB = 16
NWORKERS = NCORES * NSUB
CHUNK = 64
TILE = 4096


def _sc_gather(idx_all, B, u_comb, i_comb, w):
    per_w = B // NWORKERS
    n_chunks = per_w // CHUNK
    assert n_chunks % 2 == 0
    rows_out = jax.ShapeDtypeStruct((B, COMB), jnp.float32)
    vec_out = jax.ShapeDtypeStruct((B,), jnp.float32)
    mesh = plsc.VectorSubcoreMesh(
        core_axis_name="c", subcore_axis_name="s",
        num_cores=NCORES, num_subcores=NSUB)

    rbuf = pltpu.VMEM((CHUNK, COMB), jnp.float32)
    stage = pltpu.VMEM((per_w,), jnp.float32)

    @functools.partial(
        pl.kernel,
        out_type=[rows_out] * 3 + [vec_out] * 2,
        mesh=mesh,
        compiler_params=pltpu.CompilerParams(
            needs_layout_passes=False, use_tc_tiling_on_sc=True),
        scratch_types=[
            pltpu.VMEM((per_w,), jnp.int32),
            pltpu.VMEM((per_w,), jnp.int32),
            pltpu.VMEM((per_w,), jnp.int32),
            [rbuf] * 3,
            [rbuf] * 3,
            [stage] * 2,
            pltpu.SemaphoreType.DMA,
            pltpu.SemaphoreType.DMA,
            pltpu.SemaphoreType.DMA,
        ],
    )
    def k(idx_h, ucomb_h, icomb_h, w_h,
          o_u, o_p, o_n, o_wp, o_wn,
          idx_u, idx_p, idx_n, bufa, bufb, st, sem_g, sem_o, sem_s):
        wid = lax.axis_index("s") * NCORES + lax.axis_index("c")
        base = wid * per_w
        pltpu.sync_copy(idx_h.at[pl.ds(base, per_w)], idx_u)
        pltpu.sync_copy(idx_h.at[pl.ds(B + base, per_w)], idx_p)
        pltpu.sync_copy(idx_h.at[pl.ds(2 * B + base, per_w)], idx_n)

        pltpu.async_copy(w_h.at[idx_p.at[...]], st[0], sem_s)
        pltpu.async_copy(w_h.at[idx_n.at[...]], st[1], sem_s)

        outs = (o_u, o_p, o_n)

        def issue_gather(g, bufs):
            s = pl.ds(g * CHUNK, CHUNK)
            pltpu.async_copy(ucomb_h.at[idx_u.at[s]], bufs[0], sem_g)
            pltpu.async_copy(icomb_h.at[idx_p.at[s]], bufs[1], sem_g)
            pltpu.async_copy(icomb_h.at[idx_n.at[s]], bufs[2], sem_g)

        def wait_gather(bufs):
            s0 = pl.ds(0, CHUNK)
            pltpu.make_async_copy(
                ucomb_h.at[idx_u.at[s0]], bufs[0], sem_g).wait()
            pltpu.make_async_copy(
                icomb_h.at[idx_p.at[s0]], bufs[1], sem_g).wait()
            pltpu.make_async_copy(
                icomb_h.at[idx_n.at[s0]], bufs[2], sem_g).wait()

        def issue_out(g, bufs):
            d = pl.ds(base + g * CHUNK, CHUNK)
            for b, o in zip(bufs, outs):
                pltpu.async_copy(b, o.at[d], sem_o)

        def wait_out(bufs):
            d = pl.ds(base, CHUNK)
            for b, o in zip(bufs, outs):
                pltpu.make_async_copy(b, o.at[d], sem_o).wait()

        issue_gather(0, bufa)
        issue_gather(1, bufb)

        def pair(t, _):
            g0 = 2 * t
            g1 = 2 * t + 1
            wait_gather(bufa)
            issue_out(g0, bufa)
            wait_gather(bufb)
            issue_out(g1, bufb)
            wait_out(bufa)

            @pl.when(g0 + 2 < n_chunks)
            def _():
                issue_gather(g0 + 2, bufa)

            wait_out(bufb)

            @pl.when(g1 + 2 < n_chunks)
            def _():
                issue_gather(g1 + 2, bufb)

            return 0

        lax.fori_loop(0, n_chunks // 2, pair, 0)

        pltpu.make_async_copy(w_h.at[idx_p.at[...]], st[0], sem_s).wait()
        pltpu.make_async_copy(w_h.at[idx_n.at[...]], st[1], sem_s).wait()
        pltpu.sync_copy(st[0], o_wp.at[pl.ds(base, per_w)])
        pltpu.sync_copy(st[1], o_wn.at[pl.ds(base, per_w)])

    return k(idx_all, u_comb, i_comb, w)


def _log_sigmoid(x):
    return jnp.minimum(x, 0.0) - jnp.log1p(jnp.exp(-jnp.abs(x)))


def _softplus(x):
    return jnp.maximum(x, 0.0) + jnp.log1p(jnp.exp(-jnp.abs(x)))


def _loss_body(inv_b, u_ref, p_ref, n_ref, wp_ref, wn_ref, m_ref, o_ref):
    u = u_ref[...]
    p = p_ref[...]
    n = n_ref[...]
    pi = jnp.sum(u[:, :EMBED] * p[:, :EMBED], axis=1)
    ni = jnp.sum(u[:, :EMBED] * n[:, :EMBED], axis=1)
    pp = jnp.sum(u[:, EMBED:] * p[:, EMBED:], axis=1)
    np_ = jnp.sum(u[:, EMBED:] * n[:, EMBED:], axis=1)
    m = m_ref[...]
    loss_int = -jnp.sum(m * _log_sigmoid(pi - ni))
    loss_pop = -(jnp.sum(m * _log_sigmoid(np_ - pp))
                 + jnp.sum((1.0 - m) * _log_sigmoid(pp - np_)))
    p_tide = wp_ref[...] * (pi + pp)
    n_tide = wn_ref[...] * (ni + np_)
    loss_tide = -jnp.sum(_log_sigmoid(p_tide - n_tide))
    part = (0.1 * loss_int + 0.1 * loss_pop + 0.2 * loss_tide) * inv_b

    @pl.when(pl.program_id(0) == 0)
    def _():
        o_ref[...] = jnp.zeros((1, 1), jnp.float32)

    o_ref[...] += jnp.reshape(part, (1, 1))


def _loss_tc(urows, prows, nrows, wp, wn, maskf):
    B = wp.shape[0]
    rows_spec = pl.BlockSpec((TILE, COMB), lambda i: (i, 0))
    vec_spec = pl.BlockSpec((TILE,), lambda i: (i,))
    return pl.pallas_call(
        functools.partial(_loss_body, 1.0 / B),
        grid=(B // TILE,),
        in_specs=[rows_spec] * 3 + [vec_spec] * 3,
        out_specs=pl.BlockSpec((1, 1), lambda i: (0, 0)),
        out_shape=jax.ShapeDtypeStruct((1, 1), jnp.float32),
    )(urows, prows, nrows, wp, wn, maskf)


def kernel(user, item_p, item_n, mask, users_int, users_pop, items_int, items_pop, q, b):
    B = user.size
    idx_all = jnp.concatenate(
        [user.reshape(-1), item_p.reshape(-1), item_n.reshape(-1)])
    u_comb = jnp.concatenate([users_int, users_pop], axis=1)
    i_comb = jnp.concatenate([items_int, items_pop], axis=1)
    w = jnp.tanh(_softplus(q) + _softplus(b))
    urows, prows, nrows, wp, wn = _sc_gather(idx_all, B, u_comb, i_comb, w)
    maskf = mask.reshape(-1).astype(jnp.float32)
    loss = _loss_tc(urows, prows, nrows, wp, wn, maskf)
    return loss.reshape(())
